# baseline (speedup 1.0000x reference)
import functools

import jax
import jax.numpy as jnp
from jax import lax
from jax.experimental import pallas as pl
from jax.experimental.pallas import tpu as pltpu
from jax.experimental.pallas import tpu_sc as plsc

_LANES = 16
_NUM_DET = 100
_SEL0 = 100
_BASE = 96
_NWORK = 7
_NC = 2
_ROWS = 84
_W = _NWORK * _LANES


def _splat(cm_v, k):
    return cm_v[pl.ds(k * _LANES, _LANES)]


@functools.partial(
    pl.kernel,
    out_type=jax.ShapeDtypeStruct((_NWORK * 128,), jnp.float32),
    mesh=plsc.VectorSubcoreMesh(core_axis_name="c", subcore_axis_name="s"),
    scratch_types=[
        pltpu.VMEM((_ROWS, _LANES), jnp.float32),
        pltpu.VMEM((16 * _LANES,), jnp.float32),
        pltpu.VMEM((128,), jnp.float32),
    ],
    compiler_params=pltpu.CompilerParams(
        needs_layout_passes=False,
        skip_device_barrier=True,
        use_tc_tiling_on_sc=False,
    ),
)
def _sc_detect(x_hbm, cm_hbm, out_hbm, xv, cmv, outv):
    wid = lax.axis_index("s") * _NC + lax.axis_index("c")

    @pl.when(wid < _NWORK)
    def _():
        pltpu.sync_copy(x_hbm.at[:, pl.ds(_BASE + wid * _LANES, _LANES)], xv)
        pltpu.sync_copy(cm_hbm, cmv)

        best = xv[4, :]
        best_id = jnp.zeros((_LANES,), jnp.float32)
        for c in range(1, _ROWS - 4):
            s = xv[4 + c, :]
            pr = s > best
            best = jnp.where(pr, s, best)
            best_id = jnp.where(pr, jnp.full((_LANES,), float(c)), best_id)

        b = tuple(xv[i, :] for i in range(4))
        outv[pl.ds(0, _LANES)] = jnp.zeros((_LANES,), jnp.float32)
        for j in range(4):
            acc = b[0] * _splat(cmv, j)
            for i in range(1, 4):
                acc = acc + b[i] * _splat(cmv, i * 4 + j)
            outv[pl.ds((1 + j) * _LANES, _LANES)] = acc
        outv[pl.ds(5 * _LANES, _LANES)] = best_id
        outv[pl.ds(6 * _LANES, _LANES)] = best
        outv[pl.ds(7 * _LANES, _LANES)] = jnp.zeros((_LANES,), jnp.float32)

        pltpu.sync_copy(outv, out_hbm.at[pl.ds(wid * 128, 128)])


def kernel(x, convert_matrix):
    x2 = x.reshape(x.shape[1], x.shape[2])
    cmf = jnp.tile(convert_matrix.reshape(16, 1), (1, _LANES)).reshape(-1)
    staged = _sc_detect(x2, cmf)
    t = staged.reshape(_NWORK, 8, _LANES)[:, :7, :]
    t = jnp.transpose(t, (0, 2, 1)).reshape(_W, 7)
    off = _SEL0 - _BASE
    return t[off:off + _NUM_DET]


# speedup vs baseline: 1.0610x; 1.0610x over previous
import functools

import jax
import jax.numpy as jnp
from jax import lax
from jax.experimental import pallas as pl
from jax.experimental.pallas import tpu as pltpu
from jax.experimental.pallas import tpu_sc as plsc

_LANES = 16
_NUM_DET = 100
_SEL0 = 100
_BASE = 96
_NWORK = 7
_NC = 2
_ROWS = 84
_W = _NWORK * _LANES


def _splat(cm_v, k):
    return cm_v[pl.ds(k * _LANES, _LANES)]


@functools.partial(
    pl.kernel,
    out_type=jax.ShapeDtypeStruct((_NWORK * 128,), jnp.float32),
    mesh=plsc.VectorSubcoreMesh(
        core_axis_name="c", subcore_axis_name="s", num_cores=1),
    scratch_types=[
        pltpu.VMEM((_ROWS, _LANES), jnp.float32),
        pltpu.VMEM((16 * _LANES,), jnp.float32),
        pltpu.VMEM((128,), jnp.float32),
    ],
    compiler_params=pltpu.CompilerParams(
        needs_layout_passes=False,
        skip_device_barrier=True,
        use_tc_tiling_on_sc=False,
    ),
)
def _sc_detect(x_hbm, cm_hbm, out_hbm, xv, cmv, outv):
    wid = lax.axis_index("s")

    @pl.when(wid < _NWORK)
    def _():
        pltpu.sync_copy(x_hbm.at[:, pl.ds(_BASE + wid * _LANES, _LANES)], xv)
        pltpu.sync_copy(cm_hbm, cmv)

        def step(c, carry):
            best, best_id = carry
            s = xv[4 + c, :]
            pr = s > best
            cf = c.astype(jnp.float32)
            return (jnp.where(pr, s, best),
                    jnp.where(pr, jnp.broadcast_to(cf, (_LANES,)), best_id))

        best, best_id = lax.fori_loop(
            1, _ROWS - 4, step,
            (xv[4, :], jnp.zeros((_LANES,), jnp.float32)))

        b = tuple(xv[i, :] for i in range(4))
        outv[pl.ds(0, _LANES)] = jnp.zeros((_LANES,), jnp.float32)
        for j in range(4):
            acc = b[0] * _splat(cmv, j)
            for i in range(1, 4):
                acc = acc + b[i] * _splat(cmv, i * 4 + j)
            outv[pl.ds((1 + j) * _LANES, _LANES)] = acc
        outv[pl.ds(5 * _LANES, _LANES)] = best_id
        outv[pl.ds(6 * _LANES, _LANES)] = best
        outv[pl.ds(7 * _LANES, _LANES)] = jnp.zeros((_LANES,), jnp.float32)

        pltpu.sync_copy(outv, out_hbm.at[pl.ds(wid * 128, 128)])


def kernel(x, convert_matrix):
    x2 = x.reshape(x.shape[1], x.shape[2])
    cmf = jnp.tile(convert_matrix.reshape(16, 1), (1, _LANES)).reshape(-1)
    staged = _sc_detect(x2, cmf)
    t = staged.reshape(_NWORK, 8, _LANES)[:, :7, :]
    t = jnp.transpose(t, (0, 2, 1)).reshape(_W, 7)
    off = _SEL0 - _BASE
    return t[off:off + _NUM_DET]


# speedup vs baseline: 1.1069x; 1.0433x over previous
import functools

import jax
import jax.numpy as jnp
from jax import lax
from jax.experimental import pallas as pl
from jax.experimental.pallas import tpu as pltpu
from jax.experimental.pallas import tpu_sc as plsc

_LANES = 16
_NUM_DET = 100
_SEL0 = 100
_BASE = 96
_NWORK = 7
_ROWS = 84
_W = _NWORK * _LANES
_CM0 = _ROWS * 128


def _splat(xv, k):
    return xv[pl.ds(_CM0 + k * _LANES, _LANES)]


@functools.partial(
    pl.kernel,
    out_type=jax.ShapeDtypeStruct((_W * 8,), jnp.float32),
    mesh=plsc.VectorSubcoreMesh(
        core_axis_name="c", subcore_axis_name="s", num_cores=1),
    scratch_types=[
        pltpu.VMEM((_CM0 + 16 * _LANES,), jnp.float32),
        pltpu.VMEM((8 * _LANES,), jnp.float32),
    ],
    compiler_params=pltpu.CompilerParams(
        needs_layout_passes=False,
        skip_device_barrier=True,
    ),
)
def _sc_detect(data_hbm, out_hbm, xv, outv):
    wid = lax.axis_index("s")

    @pl.when(wid < _NWORK)
    def _():
        pltpu.sync_copy(data_hbm, xv)
        col = wid * _LANES

        def step(c, carry):
            best, best_id = carry
            s = xv[pl.ds((4 + c) * 128 + col, _LANES)]
            pr = s > best
            cf = c.astype(jnp.float32)
            return (jnp.where(pr, s, best),
                    jnp.where(pr, jnp.broadcast_to(cf, (_LANES,)), best_id))

        best, best_id = lax.fori_loop(
            1, _ROWS - 4, step,
            (xv[pl.ds(4 * 128 + col, _LANES)],
             jnp.zeros((_LANES,), jnp.float32)))

        b = tuple(xv[pl.ds(i * 128 + col, _LANES)] for i in range(4))
        lane8 = lax.iota(jnp.int32, _LANES) * 8
        zeros = jnp.zeros((_LANES,), jnp.float32)
        plsc.store_scatter(outv, [lane8], zeros)
        for j in range(4):
            acc = b[0] * _splat(xv, j)
            for i in range(1, 4):
                acc = acc + b[i] * _splat(xv, i * 4 + j)
            plsc.store_scatter(outv, [lane8 + (1 + j)], acc)
        plsc.store_scatter(outv, [lane8 + 5], best_id)
        plsc.store_scatter(outv, [lane8 + 6], best)
        plsc.store_scatter(outv, [lane8 + 7], zeros)

        pltpu.sync_copy(outv, out_hbm.at[pl.ds(wid * 8 * _LANES, 8 * _LANES)])


def kernel(x, convert_matrix):
    x2 = x.reshape(x.shape[1], x.shape[2])
    slab = x2[:, _BASE:_BASE + 128].reshape(-1)
    cmb = jnp.tile(convert_matrix.reshape(16, 1), (1, _LANES)).reshape(-1)
    staged = _sc_detect(jnp.concatenate([slab, cmb]))
    t = staged.reshape(_W, 8)
    off = _SEL0 - _BASE
    return t[off:off + _NUM_DET, :7]
